# Initial kernel scaffold; baseline (speedup 1.0000x reference)
import jax
import jax.numpy as jnp
from jax.experimental import pallas as pl


def kernel(x, codebook):
    raise NotImplementedError("write your pallas kernel here")


# speedup vs baseline: 1.8195x; 1.8195x over previous
import jax
import jax.numpy as jnp
from jax.experimental import pallas as pl

K_BINS = 1024
WIDTH = 64
TT = 512


def _vq_kernel(x_ref, cb_ref, xl_ref, xd_ref, fit_ref, sum_ref, sq_ref):
    xt = x_ref[0]
    cb = cb_ref[...]

    mm = jax.lax.dot_general(
        xt, cb, (((0,), (1,)), ((), ())),
        preferred_element_type=jnp.float32,
    )
    xsq = jnp.sum(xt * xt, axis=0)
    csq = jnp.sum(cb * cb, axis=1)
    dist = xsq[:, None] - 2.0 * mm + csq[None, :]

    minval = jnp.min(dist, axis=1)
    kiota = jax.lax.broadcasted_iota(jnp.int32, dist.shape, 1)
    idx = jnp.min(
        jnp.where(dist == minval[:, None], kiota, K_BINS), axis=1
    )
    xl_ref[0, 0, :] = idx

    onehot = (kiota == idx[:, None]).astype(jnp.float32)
    xd = jax.lax.dot_general(
        cb, onehot, (((0,), (1,)), ((), ())),
        preferred_element_type=jnp.float32,
    )
    xd_ref[0] = xd

    fit_ref[...] = jnp.sum(minval).reshape(1, 1, 1)
    sum_ref[...] = jnp.sum(xt).reshape(1, 1, 1)
    sq_ref[...] = jnp.sum(xsq).reshape(1, 1, 1)


def kernel(x, codebook):
    N, width, T = x.shape
    G = T // TT
    numel = float(N * T * width)

    out_shapes = (
        jax.ShapeDtypeStruct((N * G, 1, TT), jnp.int32),
        jax.ShapeDtypeStruct((N, width, T), jnp.float32),
        jax.ShapeDtypeStruct((N * G, 1, 1), jnp.float32),
        jax.ShapeDtypeStruct((N * G, 1, 1), jnp.float32),
        jax.ShapeDtypeStruct((N * G, 1, 1), jnp.float32),
    )
    grid = (N, G)
    xl3, x_d, fit_p, sum_p, sq_p = pl.pallas_call(
        _vq_kernel,
        grid=grid,
        in_specs=[
            pl.BlockSpec((1, width, TT), lambda i, j: (i, 0, j)),
            pl.BlockSpec((K_BINS, width), lambda i, j: (0, 0)),
        ],
        out_specs=(
            pl.BlockSpec((1, 1, TT), lambda i, j: (i * G + j, 0, 0)),
            pl.BlockSpec((1, width, TT), lambda i, j: (i, 0, j)),
            pl.BlockSpec((1, 1, 1), lambda i, j: (i * G + j, 0, 0)),
            pl.BlockSpec((1, 1, 1), lambda i, j: (i * G + j, 0, 0)),
            pl.BlockSpec((1, 1, 1), lambda i, j: (i * G + j, 0, 0)),
        ),
        out_shape=out_shapes,
    )(x, codebook)

    x_l = xl3.reshape(N, T)
    fit_sum = jnp.sum(fit_p)
    s = jnp.sum(sum_p)
    sq = jnp.sum(sq_p)

    fit = fit_sum / (N * T)
    commit_loss = fit_sum / numel
    mean = s / numel
    prenorm = jnp.sqrt(jnp.maximum(sq / numel - mean * mean, 0.0))
    return x_d, commit_loss, fit, prenorm, x_l


# speedup vs baseline: 2.0664x; 1.1357x over previous
import jax
import jax.numpy as jnp
from jax.experimental import pallas as pl

K_BINS = 1024
WIDTH = 64
TT = 512


def _vq_kernel(x_ref, cbe_ref, csq_ref, xl_ref, xd_ref, fit_ref, sum_ref, sq_ref):
    xt = x_ref[0]
    cbe = cbe_ref[...]
    csq = csq_ref[...]

    xsq = jnp.sum(xt * xt, axis=0)
    mm2 = jax.lax.dot_general(
        -2.0 * xt, cbe[:, :WIDTH], (((0,), (1,)), ((), ())),
        preferred_element_type=jnp.float32,
    )
    dist = (xsq[:, None] + mm2) + csq

    minval = jnp.min(dist, axis=1)
    onehot = (dist == minval[:, None]).astype(jnp.float32)

    res = jax.lax.dot_general(
        cbe, onehot, (((0,), (1,)), ((), ())),
        preferred_element_type=jnp.float32,
    )
    idx = res[WIDTH].astype(jnp.int32)
    cnt = res[WIDTH + 1]

    xl_ref[0, 0, :] = idx
    xd_ref[0] = res[:WIDTH]

    @pl.when(jnp.max(cnt) > 1.5)
    def _tie_fix():
        kiota = jax.lax.broadcasted_iota(jnp.int32, dist.shape, 1)
        idx2 = jnp.min(
            jnp.where(dist == minval[:, None], kiota, K_BINS), axis=1
        )
        onehot2 = (kiota == idx2[:, None]).astype(jnp.float32)
        xd2 = jax.lax.dot_general(
            cbe[:, :WIDTH], onehot2, (((0,), (1,)), ((), ())),
            preferred_element_type=jnp.float32,
        )
        xl_ref[0, 0, :] = idx2
        xd_ref[0] = xd2

    fit_ref[...] = jnp.sum(minval).reshape(1, 1, 1)
    sum_ref[...] = jnp.sum(xt).reshape(1, 1, 1)
    sq_ref[...] = jnp.sum(xsq).reshape(1, 1, 1)


def kernel(x, codebook):
    N, width, T = x.shape
    G = T // TT
    numel = float(N * T * width)

    ones_k = jnp.ones((K_BINS, 1), jnp.float32)
    iota_k = jnp.arange(K_BINS, dtype=jnp.float32)[:, None]
    cb_ext = jnp.concatenate([codebook, iota_k, ones_k], axis=1)
    csq_row = jnp.sum(codebook.T ** 2, axis=0, keepdims=True)

    out_shapes = (
        jax.ShapeDtypeStruct((N * G, 1, TT), jnp.int32),
        jax.ShapeDtypeStruct((N, width, T), jnp.float32),
        jax.ShapeDtypeStruct((N * G, 1, 1), jnp.float32),
        jax.ShapeDtypeStruct((N * G, 1, 1), jnp.float32),
        jax.ShapeDtypeStruct((N * G, 1, 1), jnp.float32),
    )
    grid = (N, G)
    xl3, x_d, fit_p, sum_p, sq_p = pl.pallas_call(
        _vq_kernel,
        grid=grid,
        in_specs=[
            pl.BlockSpec((1, width, TT), lambda i, j: (i, 0, j)),
            pl.BlockSpec((K_BINS, width + 2), lambda i, j: (0, 0)),
            pl.BlockSpec((1, K_BINS), lambda i, j: (0, 0)),
        ],
        out_specs=(
            pl.BlockSpec((1, 1, TT), lambda i, j: (i * G + j, 0, 0)),
            pl.BlockSpec((1, width, TT), lambda i, j: (i, 0, j)),
            pl.BlockSpec((1, 1, 1), lambda i, j: (i * G + j, 0, 0)),
            pl.BlockSpec((1, 1, 1), lambda i, j: (i * G + j, 0, 0)),
            pl.BlockSpec((1, 1, 1), lambda i, j: (i * G + j, 0, 0)),
        ),
        out_shape=out_shapes,
    )(x, cb_ext, csq_row)

    x_l = xl3.reshape(N, T)
    fit_sum = jnp.sum(fit_p)
    s = jnp.sum(sum_p)
    sq = jnp.sum(sq_p)

    fit = fit_sum / (N * T)
    commit_loss = fit_sum / numel
    mean = s / numel
    prenorm = jnp.sqrt(jnp.maximum(sq / numel - mean * mean, 0.0))
    return x_d, commit_loss, fit, prenorm, x_l


# speedup vs baseline: 2.6904x; 1.3020x over previous
import jax
import jax.numpy as jnp
from jax.experimental import pallas as pl

K_BINS = 1024
WIDTH = 64
TT = 1024


def _vq_kernel(x_ref, cbe_ref, csq_ref, xl_ref, xd_ref, fit_ref, sum_ref, sq_ref):
    xt = x_ref[0]
    cbe = cbe_ref[...]
    csq = csq_ref[...]

    xsq = jnp.sum(xt * xt, axis=0)
    mm2 = jax.lax.dot_general(
        -2.0 * xt, cbe[:, :WIDTH], (((0,), (1,)), ((), ())),
        preferred_element_type=jnp.float32,
    )
    dist = (xsq[:, None] + mm2) + csq

    minval = jnp.min(dist, axis=1)
    onehot = (dist == minval[:, None]).astype(jnp.float32)

    res = jax.lax.dot_general(
        cbe, onehot, (((0,), (1,)), ((), ())),
        preferred_element_type=jnp.float32,
    )
    idx = res[WIDTH].astype(jnp.int32)
    cnt = res[WIDTH + 1]

    xl_ref[0, 0, :] = idx
    xd_ref[0] = res[:WIDTH]

    @pl.when(jnp.max(cnt) > 1.5)
    def _tie_fix():
        kiota = jax.lax.broadcasted_iota(jnp.int32, dist.shape, 1)
        idx2 = jnp.min(
            jnp.where(dist == minval[:, None], kiota, K_BINS), axis=1
        )
        onehot2 = (kiota == idx2[:, None]).astype(jnp.float32)
        xd2 = jax.lax.dot_general(
            cbe[:, :WIDTH], onehot2, (((0,), (1,)), ((), ())),
            preferred_element_type=jnp.float32,
        )
        xl_ref[0, 0, :] = idx2
        xd_ref[0] = xd2

    fit_ref[...] = jnp.sum(minval).reshape(1, 1, 1)
    sum_ref[...] = jnp.sum(xt).reshape(1, 1, 1)
    sq_ref[...] = jnp.sum(xsq).reshape(1, 1, 1)


def kernel(x, codebook):
    N, width, T = x.shape
    G = T // TT
    numel = float(N * T * width)

    ones_k = jnp.ones((K_BINS, 1), jnp.float32)
    iota_k = jnp.arange(K_BINS, dtype=jnp.float32)[:, None]
    cb_ext = jnp.concatenate([codebook, iota_k, ones_k], axis=1)
    csq_row = jnp.sum(codebook.T ** 2, axis=0, keepdims=True)

    out_shapes = (
        jax.ShapeDtypeStruct((N * G, 1, TT), jnp.int32),
        jax.ShapeDtypeStruct((N, width, T), jnp.float32),
        jax.ShapeDtypeStruct((N * G, 1, 1), jnp.float32),
        jax.ShapeDtypeStruct((N * G, 1, 1), jnp.float32),
        jax.ShapeDtypeStruct((N * G, 1, 1), jnp.float32),
    )
    grid = (N, G)
    xl3, x_d, fit_p, sum_p, sq_p = pl.pallas_call(
        _vq_kernel,
        grid=grid,
        in_specs=[
            pl.BlockSpec((1, width, TT), lambda i, j: (i, 0, j)),
            pl.BlockSpec((K_BINS, width + 2), lambda i, j: (0, 0)),
            pl.BlockSpec((1, K_BINS), lambda i, j: (0, 0)),
        ],
        out_specs=(
            pl.BlockSpec((1, 1, TT), lambda i, j: (i * G + j, 0, 0)),
            pl.BlockSpec((1, width, TT), lambda i, j: (i, 0, j)),
            pl.BlockSpec((1, 1, 1), lambda i, j: (i * G + j, 0, 0)),
            pl.BlockSpec((1, 1, 1), lambda i, j: (i * G + j, 0, 0)),
            pl.BlockSpec((1, 1, 1), lambda i, j: (i * G + j, 0, 0)),
        ),
        out_shape=out_shapes,
    )(x, cb_ext, csq_row)

    x_l = xl3.reshape(N, T)
    fit_sum = jnp.sum(fit_p)
    s = jnp.sum(sum_p)
    sq = jnp.sum(sq_p)

    fit = fit_sum / (N * T)
    commit_loss = fit_sum / numel
    mean = s / numel
    prenorm = jnp.sqrt(jnp.maximum(sq / numel - mean * mean, 0.0))
    return x_d, commit_loss, fit, prenorm, x_l


# speedup vs baseline: 3.3593x; 1.2486x over previous
import jax
import jax.numpy as jnp
from jax.experimental import pallas as pl

K_BINS = 1024
WIDTH = 64
TT = 2048


def _vq_kernel(x_ref, cbe_ref, csq_ref, xl_ref, xd_ref, fit_ref, sum_ref, sq_ref):
    xt = x_ref[0]
    cbe = cbe_ref[...]
    csq = csq_ref[...]

    xsq = jnp.sum(xt * xt, axis=0)
    mm2 = jax.lax.dot_general(
        -2.0 * xt, cbe[:, :WIDTH], (((0,), (1,)), ((), ())),
        preferred_element_type=jnp.float32,
    )
    dist = (xsq[:, None] + mm2) + csq

    minval = jnp.min(dist, axis=1)
    onehot = (dist == minval[:, None]).astype(jnp.float32)

    res = jax.lax.dot_general(
        cbe, onehot, (((0,), (1,)), ((), ())),
        preferred_element_type=jnp.float32,
    )
    idx = res[WIDTH].astype(jnp.int32)
    cnt = res[WIDTH + 1]

    xl_ref[0, 0, :] = idx
    xd_ref[0] = res[:WIDTH]

    @pl.when(jnp.max(cnt) > 1.5)
    def _tie_fix():
        kiota = jax.lax.broadcasted_iota(jnp.int32, dist.shape, 1)
        idx2 = jnp.min(
            jnp.where(dist == minval[:, None], kiota, K_BINS), axis=1
        )
        onehot2 = (kiota == idx2[:, None]).astype(jnp.float32)
        xd2 = jax.lax.dot_general(
            cbe[:, :WIDTH], onehot2, (((0,), (1,)), ((), ())),
            preferred_element_type=jnp.float32,
        )
        xl_ref[0, 0, :] = idx2
        xd_ref[0] = xd2

    fit_ref[...] = jnp.sum(minval).reshape(1, 1, 1)
    sum_ref[...] = jnp.sum(xt).reshape(1, 1, 1)
    sq_ref[...] = jnp.sum(xsq).reshape(1, 1, 1)


def kernel(x, codebook):
    N, width, T = x.shape
    G = T // TT
    numel = float(N * T * width)

    ones_k = jnp.ones((K_BINS, 1), jnp.float32)
    iota_k = jnp.arange(K_BINS, dtype=jnp.float32)[:, None]
    cb_ext = jnp.concatenate([codebook, iota_k, ones_k], axis=1)
    csq_row = jnp.sum(codebook.T ** 2, axis=0, keepdims=True)

    out_shapes = (
        jax.ShapeDtypeStruct((N * G, 1, TT), jnp.int32),
        jax.ShapeDtypeStruct((N, width, T), jnp.float32),
        jax.ShapeDtypeStruct((N * G, 1, 1), jnp.float32),
        jax.ShapeDtypeStruct((N * G, 1, 1), jnp.float32),
        jax.ShapeDtypeStruct((N * G, 1, 1), jnp.float32),
    )
    grid = (N, G)
    xl3, x_d, fit_p, sum_p, sq_p = pl.pallas_call(
        _vq_kernel,
        grid=grid,
        in_specs=[
            pl.BlockSpec((1, width, TT), lambda i, j: (i, 0, j)),
            pl.BlockSpec((K_BINS, width + 2), lambda i, j: (0, 0)),
            pl.BlockSpec((1, K_BINS), lambda i, j: (0, 0)),
        ],
        out_specs=(
            pl.BlockSpec((1, 1, TT), lambda i, j: (i * G + j, 0, 0)),
            pl.BlockSpec((1, width, TT), lambda i, j: (i, 0, j)),
            pl.BlockSpec((1, 1, 1), lambda i, j: (i * G + j, 0, 0)),
            pl.BlockSpec((1, 1, 1), lambda i, j: (i * G + j, 0, 0)),
            pl.BlockSpec((1, 1, 1), lambda i, j: (i * G + j, 0, 0)),
        ),
        out_shape=out_shapes,
    )(x, cb_ext, csq_row)

    x_l = xl3.reshape(N, T)
    fit_sum = jnp.sum(fit_p)
    s = jnp.sum(sum_p)
    sq = jnp.sum(sq_p)

    fit = fit_sum / (N * T)
    commit_loss = fit_sum / numel
    mean = s / numel
    prenorm = jnp.sqrt(jnp.maximum(sq / numel - mean * mean, 0.0))
    return x_d, commit_loss, fit, prenorm, x_l


# speedup vs baseline: 3.5199x; 1.0478x over previous
import jax
import jax.numpy as jnp
from jax.experimental import pallas as pl

K_BINS = 1024
WIDTH = 64
TT = 4096


def _vq_kernel(x_ref, cbe_ref, csq_ref, xl_ref, xd_ref, fit_ref, sum_ref, sq_ref):
    xt = x_ref[0]
    cbe = cbe_ref[...]
    csq = csq_ref[...]

    xsq = jnp.sum(xt * xt, axis=0)
    mm2 = jax.lax.dot_general(
        -2.0 * xt, cbe[:, :WIDTH], (((0,), (1,)), ((), ())),
        preferred_element_type=jnp.float32,
    )
    dist = (xsq[:, None] + mm2) + csq

    minval = jnp.min(dist, axis=1)
    onehot = (dist == minval[:, None]).astype(jnp.float32)

    res = jax.lax.dot_general(
        cbe, onehot, (((0,), (1,)), ((), ())),
        preferred_element_type=jnp.float32,
    )
    idx = res[WIDTH].astype(jnp.int32)
    cnt = res[WIDTH + 1]

    xl_ref[0, 0, :] = idx
    xd_ref[0] = res[:WIDTH]

    @pl.when(jnp.max(cnt) > 1.5)
    def _tie_fix():
        kiota = jax.lax.broadcasted_iota(jnp.int32, dist.shape, 1)
        idx2 = jnp.min(
            jnp.where(dist == minval[:, None], kiota, K_BINS), axis=1
        )
        onehot2 = (kiota == idx2[:, None]).astype(jnp.float32)
        xd2 = jax.lax.dot_general(
            cbe[:, :WIDTH], onehot2, (((0,), (1,)), ((), ())),
            preferred_element_type=jnp.float32,
        )
        xl_ref[0, 0, :] = idx2
        xd_ref[0] = xd2

    fit_ref[...] = jnp.sum(minval).reshape(1, 1, 1)
    sum_ref[...] = jnp.sum(xt).reshape(1, 1, 1)
    sq_ref[...] = jnp.sum(xsq).reshape(1, 1, 1)


def kernel(x, codebook):
    N, width, T = x.shape
    G = T // TT
    numel = float(N * T * width)

    ones_k = jnp.ones((K_BINS, 1), jnp.float32)
    iota_k = jnp.arange(K_BINS, dtype=jnp.float32)[:, None]
    cb_ext = jnp.concatenate([codebook, iota_k, ones_k], axis=1)
    csq_row = jnp.sum(codebook.T ** 2, axis=0, keepdims=True)

    out_shapes = (
        jax.ShapeDtypeStruct((N * G, 1, TT), jnp.int32),
        jax.ShapeDtypeStruct((N, width, T), jnp.float32),
        jax.ShapeDtypeStruct((N * G, 1, 1), jnp.float32),
        jax.ShapeDtypeStruct((N * G, 1, 1), jnp.float32),
        jax.ShapeDtypeStruct((N * G, 1, 1), jnp.float32),
    )
    grid = (N, G)
    xl3, x_d, fit_p, sum_p, sq_p = pl.pallas_call(
        _vq_kernel,
        grid=grid,
        in_specs=[
            pl.BlockSpec((1, width, TT), lambda i, j: (i, 0, j)),
            pl.BlockSpec((K_BINS, width + 2), lambda i, j: (0, 0)),
            pl.BlockSpec((1, K_BINS), lambda i, j: (0, 0)),
        ],
        out_specs=(
            pl.BlockSpec((1, 1, TT), lambda i, j: (i * G + j, 0, 0)),
            pl.BlockSpec((1, width, TT), lambda i, j: (i, 0, j)),
            pl.BlockSpec((1, 1, 1), lambda i, j: (i * G + j, 0, 0)),
            pl.BlockSpec((1, 1, 1), lambda i, j: (i * G + j, 0, 0)),
            pl.BlockSpec((1, 1, 1), lambda i, j: (i * G + j, 0, 0)),
        ),
        out_shape=out_shapes,
    )(x, cb_ext, csq_row)

    x_l = xl3.reshape(N, T)
    fit_sum = jnp.sum(fit_p)
    s = jnp.sum(sum_p)
    sq = jnp.sum(sq_p)

    fit = fit_sum / (N * T)
    commit_loss = fit_sum / numel
    mean = s / numel
    prenorm = jnp.sqrt(jnp.maximum(sq / numel - mean * mean, 0.0))
    return x_d, commit_loss, fit, prenorm, x_l
